# Initial kernel scaffold; baseline (speedup 1.0000x reference)
import jax
import jax.numpy as jnp
from jax.experimental import pallas as pl


def kernel(x, pos_emb):
    raise NotImplementedError("write your pallas kernel here")


# speedup vs baseline: 7.1620x; 7.1620x over previous
import jax
import jax.numpy as jnp
from jax.experimental import pallas as pl
from jax.experimental.pallas import tpu as pltpu

_BB = 128


def _body(xt_ref, body_ref, row0_ref, o_ref):
    L, D = body_ref.shape[1], body_ref.shape[2]
    body = body_ref[0]
    row0 = jnp.broadcast_to(row0_ref[0], (L, D))
    for b in range(o_ref.shape[0]):
        mask = xt_ref[:, b:b + 1] != 0
        mask = jnp.broadcast_to(mask, (L, D))
        o_ref[b] = jnp.where(mask, body, row0)


def kernel(x, pos_emb):
    B, L = x.shape
    D = pos_emb.shape[1]
    xt = x.T
    body = pos_emb[1:L + 1].reshape(1, L, D)
    row0 = pos_emb[0:1].reshape(1, 1, D)
    grid = (B // _BB,)
    return pl.pallas_call(
        _body,
        grid=grid,
        in_specs=[
            pl.BlockSpec((L, _BB), lambda i: (0, i)),
            pl.BlockSpec((1, L, D), lambda i: (0, 0, 0)),
            pl.BlockSpec((1, 1, D), lambda i: (0, 0, 0)),
        ],
        out_specs=pl.BlockSpec((_BB, L, D), lambda i: (i, 0, 0)),
        out_shape=jax.ShapeDtypeStruct((B, L, D), jnp.float32),
        compiler_params=pltpu.CompilerParams(
            dimension_semantics=("arbitrary",),
        ),
    )(xt, body, row0)
